# baseline (speedup 1.0000x reference)
import functools

import jax
import jax.numpy as jnp
from jax import lax
from jax.experimental import pallas as pl
from jax.experimental.pallas import tpu as pltpu
from jax.experimental.pallas import tpu_sc as plsc

NC = 2
NS = 16
NW = NC * NS
K = 128
DEGW = 128


def _sc_mesh():
    return plsc.VectorSubcoreMesh(
        core_axis_name="c", subcore_axis_name="s",
        num_cores=NC, num_subcores=NS)


def _make_deg(N_pad, E_pad):
    epw = E_pad // NW
    nch = epw // K
    rpw = N_pad // NS

    @functools.partial(
        pl.kernel,
        out_type=jax.ShapeDtypeStruct((NC, N_pad, DEGW), jnp.float32),
        mesh=_sc_mesh(),
        scratch_types=[
            pltpu.VMEM((nch, K), jnp.int32),
            pltpu.VMEM((K, DEGW), jnp.float32),
            pltpu.VMEM_SHARED((N_pad, DEGW), jnp.float32),
        ],
    )
    def deg_kernel(dst_hbm, ones_hbm, zeros_hbm, out_hbm, dstv, ones_v, acc_sh):
        c = lax.axis_index("c")
        s = lax.axis_index("s")
        wid = c * NS + s
        pltpu.sync_copy(ones_hbm, ones_v)
        pltpu.sync_copy(dst_hbm.at[pl.ds(wid * nch, nch)], dstv)
        pltpu.sync_copy(zeros_hbm, acc_sh.at[pl.ds(s * rpw, rpw)])
        plsc.subcore_barrier()

        def chunk(i, carry):
            pltpu.sync_copy(ones_v, acc_sh.at[dstv.at[i]], add=True)
            return carry

        lax.fori_loop(0, nch, chunk, 0)
        plsc.subcore_barrier()
        pltpu.sync_copy(acc_sh.at[pl.ds(s * rpw, rpw)],
                        out_hbm.at[c, pl.ds(s * rpw, rpw)])

    return deg_kernel


def _make_agg(N_pad, E_pad, D):
    epw = E_pad // NW
    nch = epw // K
    rpw = N_pad // NS

    npair = nch // 2

    @functools.partial(
        pl.kernel,
        out_type=jax.ShapeDtypeStruct((NC, N_pad, D), jnp.float32),
        mesh=_sc_mesh(),
        scratch_types=[
            pltpu.VMEM((nch + 1, K), jnp.int32),
            pltpu.VMEM((nch, K), jnp.int32),
            pltpu.VMEM((K, D), jnp.float32),
            pltpu.VMEM((K, D), jnp.float32),
            pltpu.VMEM_SHARED((N_pad, D), jnp.float32),
            pltpu.SemaphoreType.DMA,
            pltpu.SemaphoreType.DMA,
        ],
    )
    def agg_kernel(hs_hbm, src_hbm, dst_hbm, zeros_hbm, out_hbm,
                   srcv, dstv, buf0, buf1, acc_sh, sem0, sem1):
        c = lax.axis_index("c")
        s = lax.axis_index("s")
        wid = c * NS + s
        pltpu.sync_copy(src_hbm.at[pl.ds(wid * nch, nch)],
                        srcv.at[pl.ds(0, nch)])
        pltpu.sync_copy(dst_hbm.at[pl.ds(wid * nch, nch)], dstv)
        z16 = jnp.zeros((16,), jnp.int32)
        for j in range(K // 16):
            srcv[nch, pl.ds(j * 16, 16)] = z16
        pltpu.sync_copy(zeros_hbm, acc_sh.at[pl.ds(s * rpw, rpw)])
        plsc.subcore_barrier()

        def drain(buf, sem):
            pltpu.make_async_copy(hs_hbm.at[pl.ds(0, K)], buf, sem).wait()

        pltpu.async_copy(hs_hbm.at[srcv.at[0]], buf0, sem0)

        def pair(p, carry):
            i0 = 2 * p
            pltpu.async_copy(hs_hbm.at[srcv.at[i0 + 1]], buf1, sem1)
            drain(buf0, sem0)
            pltpu.sync_copy(buf0, acc_sh.at[dstv.at[i0]], add=True)
            pltpu.async_copy(hs_hbm.at[srcv.at[i0 + 2]], buf0, sem0)
            drain(buf1, sem1)
            pltpu.sync_copy(buf1, acc_sh.at[dstv.at[i0 + 1]], add=True)
            return carry

        lax.fori_loop(0, npair, pair, 0)
        drain(buf0, sem0)
        plsc.subcore_barrier()
        pltpu.sync_copy(acc_sh.at[pl.ds(s * rpw, rpw)],
                        out_hbm.at[c, pl.ds(s * rpw, rpw)])

    return agg_kernel


def _row_spec(R, D):
    return pl.BlockSpec((R, D), lambda i: (i, 0))


def _full_spec(shape):
    nd = len(shape)
    return pl.BlockSpec(shape, lambda i, _nd=nd: (0,) * _nd)


def _scale_from_deg(degp_ref):
    deg = degp_ref[0, :, 0:1] + degp_ref[1, :, 0:1]
    return lax.rsqrt(deg + 1.0)


def _enc_call(features, degp, W_enc, b_enc, R):
    N, D_in = features.shape
    D = W_enc.shape[1]

    def body(x_ref, degp_ref, w_ref, b_ref, h_ref, hs_ref):
        h = jnp.dot(x_ref[...], w_ref[...],
                    preferred_element_type=jnp.float32) + b_ref[...]
        s = _scale_from_deg(degp_ref)
        h_ref[...] = h
        hs_ref[...] = h * s

    return pl.pallas_call(
        body,
        grid=(N // R,),
        in_specs=[
            _row_spec(R, D_in),
            pl.BlockSpec((NC, R, DEGW), lambda i: (0, i, 0)),
            _full_spec((D_in, D)),
            _full_spec((1, D)),
        ],
        out_specs=[_row_spec(R, D), _row_spec(R, D)],
        out_shape=[jax.ShapeDtypeStruct((N, D), jnp.float32)] * 2,
    )(features, degp, W_enc, b_enc[None, :])


def _comb_call(h, aggp, degp, W_top, W_bot, b, R):
    N, D = h.shape

    def body(h_ref, aggp_ref, degp_ref, wt_ref, wb_ref, b_ref,
             h2_ref, h2s_ref):
        s = _scale_from_deg(degp_ref)
        agg = (aggp_ref[0] + aggp_ref[1]) * s
        z = (jnp.dot(h_ref[...], wt_ref[...], preferred_element_type=jnp.float32)
             + jnp.dot(agg, wb_ref[...], preferred_element_type=jnp.float32)
             + b_ref[...])
        h2 = jnp.maximum(z, 0.0)
        h2_ref[...] = h2
        h2s_ref[...] = h2 * s

    return pl.pallas_call(
        body,
        grid=(N // R,),
        in_specs=[
            _row_spec(R, D),
            pl.BlockSpec((NC, R, D), lambda i: (0, i, 0)),
            pl.BlockSpec((NC, R, DEGW), lambda i: (0, i, 0)),
            _full_spec((D, D)),
            _full_spec((D, D)),
            _full_spec((1, D)),
        ],
        out_specs=[_row_spec(R, D), _row_spec(R, D)],
        out_shape=[jax.ShapeDtypeStruct((N, D), jnp.float32)] * 2,
    )(h, aggp, degp, W_top, W_bot, b[None, :])


def _comb_head_call(h, aggp, degp, W_top, W_bot, b, W_out, b_out, R):
    N, D = h.shape
    D_out = W_out.shape[1]

    def body(h_ref, aggp_ref, degp_ref, wt_ref, wb_ref, b_ref,
             wo_ref, bo_ref, out_ref):
        s = _scale_from_deg(degp_ref)
        agg = (aggp_ref[0] + aggp_ref[1]) * s
        z = (jnp.dot(h_ref[...], wt_ref[...], preferred_element_type=jnp.float32)
             + jnp.dot(agg, wb_ref[...], preferred_element_type=jnp.float32)
             + b_ref[...])
        h2 = jnp.maximum(z, 0.0)
        out_ref[...] = jnp.dot(h2, wo_ref[...],
                               preferred_element_type=jnp.float32) + bo_ref[...]

    return pl.pallas_call(
        body,
        grid=(N // R,),
        in_specs=[
            _row_spec(R, D),
            pl.BlockSpec((NC, R, D), lambda i: (0, i, 0)),
            pl.BlockSpec((NC, R, DEGW), lambda i: (0, i, 0)),
            _full_spec((D, D)),
            _full_spec((D, D)),
            _full_spec((1, D)),
            _full_spec((D, D_out)),
            _full_spec((1, D_out)),
        ],
        out_specs=pl.BlockSpec((R, D_out), lambda i: (i, 0)),
        out_shape=jax.ShapeDtypeStruct((N, D_out), jnp.float32),
    )(h, aggp, degp, W_top, W_bot, b[None, :], W_out, b_out[None, :])


def kernel(features, edge_list, W_enc, b_enc, W_comb, b_comb, W_out, b_out):
    N, D_in = features.shape
    E = edge_list.shape[0]
    D = W_enc.shape[1]
    layers = W_comb.shape[0]

    R = 1000
    N_pad = ((N + 16 * NS - 1) // (16 * NS)) * (16 * NS)
    E_pad = ((E + NW * K - 1) // (NW * K)) * (NW * K)

    src = edge_list[:, 0]
    dst = edge_list[:, 1]
    pad = E_pad - E
    src_p = jnp.concatenate([src, jnp.zeros((pad,), jnp.int32)])
    dst_p = jnp.concatenate([dst, jnp.full((pad,), N, jnp.int32)])
    src_p = src_p.reshape(E_pad // K, K)
    dst_p = dst_p.reshape(E_pad // K, K)

    rpw = N_pad // NS
    zeros_deg = jnp.zeros((rpw, DEGW), jnp.float32)
    ones_deg = jnp.ones((K, DEGW), jnp.float32)
    zeros_agg = jnp.zeros((rpw, D), jnp.float32)

    degp = _make_deg(N_pad, E_pad)(dst_p, ones_deg, zeros_deg)
    h, hs = _enc_call(features, degp, W_enc, b_enc, R)
    agg_fn = _make_agg(N_pad, E_pad, D)
    for l in range(layers - 1):
        aggp = agg_fn(hs, src_p, dst_p, zeros_agg)
        h, hs = _comb_call(h, aggp, degp,
                           W_comb[l, :D], W_comb[l, D:], b_comb[l], R)
    aggp = agg_fn(hs, src_p, dst_p, zeros_agg)
    return _comb_head_call(h, aggp, degp,
                           W_comb[layers - 1, :D], W_comb[layers - 1, D:],
                           b_comb[layers - 1], W_out, b_out, R)


# speedup vs baseline: 1.0004x; 1.0004x over previous
import functools

import jax
import jax.numpy as jnp
from jax import lax
from jax.experimental import pallas as pl
from jax.experimental.pallas import tpu as pltpu
from jax.experimental.pallas import tpu_sc as plsc

NC = 2
NS = 16
NW = NC * NS
K = 128
DEGW = 128


def _sc_mesh():
    return plsc.VectorSubcoreMesh(
        core_axis_name="c", subcore_axis_name="s",
        num_cores=NC, num_subcores=NS)


def _make_deg(N_pad, E_pad):
    epw = E_pad // NW
    nch = epw // K
    rpw = N_pad // NS

    @functools.partial(
        pl.kernel,
        out_type=jax.ShapeDtypeStruct((NC, N_pad, DEGW), jnp.float32),
        mesh=_sc_mesh(),
        scratch_types=[
            pltpu.VMEM((nch, K), jnp.int32),
            pltpu.VMEM((K, DEGW), jnp.float32),
            pltpu.VMEM_SHARED((N_pad, DEGW), jnp.float32),
        ],
    )
    def deg_kernel(dst_hbm, ones_hbm, zeros_hbm, out_hbm, dstv, ones_v, acc_sh):
        c = lax.axis_index("c")
        s = lax.axis_index("s")
        wid = c * NS + s
        pltpu.sync_copy(ones_hbm, ones_v)
        pltpu.sync_copy(dst_hbm.at[pl.ds(wid * nch, nch)], dstv)
        pltpu.sync_copy(zeros_hbm, acc_sh.at[pl.ds(s * rpw, rpw)])
        plsc.subcore_barrier()

        def chunk(i, carry):
            pltpu.sync_copy(ones_v, acc_sh.at[dstv.at[i]], add=True)
            return carry

        lax.fori_loop(0, nch, chunk, 0)
        plsc.subcore_barrier()
        pltpu.sync_copy(acc_sh.at[pl.ds(s * rpw, rpw)],
                        out_hbm.at[c, pl.ds(s * rpw, rpw)])

    return deg_kernel


def _make_agg(N_pad, E_pad, D):
    epw = E_pad // NW
    nch = epw // K
    rpw = N_pad // NS

    npair = nch // 2

    @functools.partial(
        pl.kernel,
        out_type=jax.ShapeDtypeStruct((NC, N_pad, D), jnp.float32),
        mesh=_sc_mesh(),
        scratch_types=[
            pltpu.VMEM((nch + 1, K), jnp.int32),
            pltpu.VMEM((nch, K), jnp.int32),
            pltpu.VMEM((K, D), jnp.float32),
            pltpu.VMEM((K, D), jnp.float32),
            pltpu.VMEM_SHARED((N_pad, D), jnp.float32),
            pltpu.SemaphoreType.DMA,
            pltpu.SemaphoreType.DMA,
        ],
    )
    def agg_kernel(hs_hbm, src_hbm, dst_hbm, zeros_hbm, out_hbm,
                   srcv, dstv, buf0, buf1, acc_sh, sem0, sem1):
        c = lax.axis_index("c")
        s = lax.axis_index("s")
        wid = c * NS + s
        pltpu.sync_copy(src_hbm.at[pl.ds(wid * nch, nch)],
                        srcv.at[pl.ds(0, nch)])
        pltpu.sync_copy(dst_hbm.at[pl.ds(wid * nch, nch)], dstv)
        z16 = jnp.zeros((16,), jnp.int32)
        for j in range(K // 16):
            srcv[nch, pl.ds(j * 16, 16)] = z16
        pltpu.sync_copy(zeros_hbm, acc_sh.at[pl.ds(s * rpw, rpw)])
        plsc.subcore_barrier()

        def drain(buf, sem):
            pltpu.make_async_copy(hs_hbm.at[pl.ds(0, K)], buf, sem).wait()

        pltpu.async_copy(hs_hbm.at[srcv.at[0]], buf0, sem0)

        def pair(p, carry):
            i0 = 2 * p
            pltpu.async_copy(hs_hbm.at[srcv.at[i0 + 1]], buf1, sem1)
            drain(buf0, sem0)
            pltpu.sync_copy(buf0, acc_sh.at[dstv.at[i0]], add=True)
            pltpu.async_copy(hs_hbm.at[srcv.at[i0 + 2]], buf0, sem0)
            drain(buf1, sem1)
            pltpu.sync_copy(buf1, acc_sh.at[dstv.at[i0 + 1]], add=True)
            return carry

        lax.fori_loop(0, npair, pair, 0)
        drain(buf0, sem0)
        plsc.subcore_barrier()
        pltpu.sync_copy(acc_sh.at[pl.ds(s * rpw, rpw)],
                        out_hbm.at[c, pl.ds(s * rpw, rpw)])

    return agg_kernel


def _row_spec(R, D):
    return pl.BlockSpec((R, D), lambda i: (i, 0))


def _full_spec(shape):
    nd = len(shape)
    return pl.BlockSpec(shape, lambda i, _nd=nd: (0,) * _nd)


def _scale_from_deg(degp_ref):
    deg = degp_ref[0, :, 0:1] + degp_ref[1, :, 0:1]
    return lax.rsqrt(deg + 1.0)


def _enc_call(features, degp, W_enc, b_enc, R):
    N, D_in = features.shape
    D = W_enc.shape[1]

    def body(x_ref, degp_ref, w_ref, b_ref, h_ref, hs_ref):
        h = jnp.dot(x_ref[...], w_ref[...],
                    preferred_element_type=jnp.float32) + b_ref[...]
        s = _scale_from_deg(degp_ref)
        h_ref[...] = h
        hs_ref[...] = h * s

    return pl.pallas_call(
        body,
        grid=(N // R,),
        in_specs=[
            _row_spec(R, D_in),
            pl.BlockSpec((NC, R, DEGW), lambda i: (0, i, 0)),
            _full_spec((D_in, D)),
            _full_spec((1, D)),
        ],
        out_specs=[_row_spec(R, D), _row_spec(R, D)],
        out_shape=[jax.ShapeDtypeStruct((N, D), jnp.float32)] * 2,
    )(features, degp, W_enc, b_enc[None, :])


def _comb_call(h, aggp, degp, W_top, W_bot, b, R):
    N, D = h.shape

    def body(h_ref, aggp_ref, degp_ref, wt_ref, wb_ref, b_ref,
             h2_ref, h2s_ref):
        s = _scale_from_deg(degp_ref)
        agg = (aggp_ref[0] + aggp_ref[1]) * s
        z = (jnp.dot(h_ref[...], wt_ref[...], preferred_element_type=jnp.float32)
             + jnp.dot(agg, wb_ref[...], preferred_element_type=jnp.float32)
             + b_ref[...])
        h2 = jnp.maximum(z, 0.0)
        h2_ref[...] = h2
        h2s_ref[...] = h2 * s

    return pl.pallas_call(
        body,
        grid=(N // R,),
        in_specs=[
            _row_spec(R, D),
            pl.BlockSpec((NC, R, D), lambda i: (0, i, 0)),
            pl.BlockSpec((NC, R, DEGW), lambda i: (0, i, 0)),
            _full_spec((D, D)),
            _full_spec((D, D)),
            _full_spec((1, D)),
        ],
        out_specs=[_row_spec(R, D), _row_spec(R, D)],
        out_shape=[jax.ShapeDtypeStruct((N, D), jnp.float32)] * 2,
    )(h, aggp, degp, W_top, W_bot, b[None, :])


def _comb_head_call(h, aggp, degp, W_top, W_bot, b, W_out, b_out, R):
    N, D = h.shape
    D_out = W_out.shape[1]

    def body(h_ref, aggp_ref, degp_ref, wt_ref, wb_ref, b_ref,
             wo_ref, bo_ref, out_ref):
        s = _scale_from_deg(degp_ref)
        agg = (aggp_ref[0] + aggp_ref[1]) * s
        z = (jnp.dot(h_ref[...], wt_ref[...], preferred_element_type=jnp.float32)
             + jnp.dot(agg, wb_ref[...], preferred_element_type=jnp.float32)
             + b_ref[...])
        h2 = jnp.maximum(z, 0.0)
        out_ref[...] = jnp.dot(h2, wo_ref[...],
                               preferred_element_type=jnp.float32) + bo_ref[...]

    return pl.pallas_call(
        body,
        grid=(N // R,),
        in_specs=[
            _row_spec(R, D),
            pl.BlockSpec((NC, R, D), lambda i: (0, i, 0)),
            pl.BlockSpec((NC, R, DEGW), lambda i: (0, i, 0)),
            _full_spec((D, D)),
            _full_spec((D, D)),
            _full_spec((1, D)),
            _full_spec((D, D_out)),
            _full_spec((1, D_out)),
        ],
        out_specs=pl.BlockSpec((R, D_out), lambda i: (i, 0)),
        out_shape=jax.ShapeDtypeStruct((N, D_out), jnp.float32),
    )(h, aggp, degp, W_top, W_bot, b[None, :], W_out, b_out[None, :])


def kernel(features, edge_list, W_enc, b_enc, W_comb, b_comb, W_out, b_out):
    N, D_in = features.shape
    E = edge_list.shape[0]
    D = W_enc.shape[1]
    layers = W_comb.shape[0]

    R = 1000
    N_pad = ((N + 16 * NS - 1) // (16 * NS)) * (16 * NS)
    E_pad = ((E + NW * K - 1) // (NW * K)) * (NW * K)

    src = edge_list[:, 0]
    dst = edge_list[:, 1]
    pad = E_pad - E
    src_p = jnp.concatenate([src, jnp.zeros((pad,), jnp.int32)])
    trash = N + jnp.arange(pad, dtype=jnp.int32) % (N_pad - N)
    dst_p = jnp.concatenate([dst, trash])
    src_p = src_p.reshape(E_pad // K, K)
    dst_p = dst_p.reshape(E_pad // K, K)

    rpw = N_pad // NS
    zeros_deg = jnp.zeros((rpw, DEGW), jnp.float32)
    ones_deg = jnp.ones((K, DEGW), jnp.float32)
    zeros_agg = jnp.zeros((rpw, D), jnp.float32)

    degp = _make_deg(N_pad, E_pad)(dst_p, ones_deg, zeros_deg)
    h, hs = _enc_call(features, degp, W_enc, b_enc, R)
    agg_fn = _make_agg(N_pad, E_pad, D)
    for l in range(layers - 1):
        aggp = agg_fn(hs, src_p, dst_p, zeros_agg)
        h, hs = _comb_call(h, aggp, degp,
                           W_comb[l, :D], W_comb[l, D:], b_comb[l], R)
    aggp = agg_fn(hs, src_p, dst_p, zeros_agg)
    return _comb_head_call(h, aggp, degp,
                           W_comb[layers - 1, :D], W_comb[layers - 1, D:],
                           b_comb[layers - 1], W_out, b_out, R)


# speedup vs baseline: 1.3857x; 1.3852x over previous
import functools

import jax
import jax.numpy as jnp
from jax import lax
from jax.experimental import pallas as pl
from jax.experimental.pallas import tpu as pltpu
from jax.experimental.pallas import tpu_sc as plsc

NC = 2
NS = 16
NW = NC * NS
K = 128
DEGW = 128


def _sc_mesh():
    return plsc.VectorSubcoreMesh(
        core_axis_name="c", subcore_axis_name="s",
        num_cores=NC, num_subcores=NS)


def _make_deg(N_pad, E_pad):
    epw = E_pad // NW
    nch = epw // K
    rpw = N_pad // NS

    @functools.partial(
        pl.kernel,
        out_type=jax.ShapeDtypeStruct((NC, N_pad, DEGW), jnp.float32),
        mesh=_sc_mesh(),
        scratch_types=[
            pltpu.VMEM((nch, K), jnp.int32),
            pltpu.VMEM((K, DEGW), jnp.float32),
            pltpu.VMEM_SHARED((N_pad, DEGW), jnp.float32),
        ],
    )
    def deg_kernel(dst_hbm, ones_hbm, zeros_hbm, out_hbm, dstv, ones_v, acc_sh):
        c = lax.axis_index("c")
        s = lax.axis_index("s")
        wid = c * NS + s
        pltpu.sync_copy(ones_hbm, ones_v)
        pltpu.sync_copy(dst_hbm.at[pl.ds(wid * nch, nch)], dstv)
        pltpu.sync_copy(zeros_hbm, acc_sh.at[pl.ds(s * rpw, rpw)])
        plsc.subcore_barrier()

        def chunk(i, carry):
            pltpu.sync_copy(ones_v, acc_sh.at[dstv.at[i]], add=True)
            return carry

        lax.fori_loop(0, nch, chunk, 0)
        plsc.subcore_barrier()
        pltpu.sync_copy(acc_sh.at[pl.ds(s * rpw, rpw)],
                        out_hbm.at[c, pl.ds(s * rpw, rpw)])

    return deg_kernel


def _make_agg(N_pad, E_pad, D):
    epw = E_pad // NW
    nch = epw // K
    rpw = N_pad // NS

    npair = nch // 2

    @functools.partial(
        pl.kernel,
        out_type=jax.ShapeDtypeStruct((NC, N_pad, D), jnp.float32),
        mesh=_sc_mesh(),
        scratch_types=[
            pltpu.VMEM((nch + 1, K), jnp.int32),
            pltpu.VMEM((nch, K), jnp.int32),
            pltpu.VMEM((K, D), jnp.float32),
            pltpu.VMEM((K, D), jnp.float32),
            pltpu.VMEM_SHARED((N_pad, D), jnp.float32),
            pltpu.SemaphoreType.DMA,
            pltpu.SemaphoreType.DMA,
        ],
    )
    def agg_kernel(hs_hbm, src_hbm, dst_hbm, zeros_hbm, out_hbm,
                   srcv, dstv, buf0, buf1, acc_sh, sem0, sem1):
        c = lax.axis_index("c")
        s = lax.axis_index("s")
        wid = c * NS + s
        pltpu.sync_copy(src_hbm.at[pl.ds(wid * nch, nch)],
                        srcv.at[pl.ds(0, nch)])
        pltpu.sync_copy(dst_hbm.at[pl.ds(wid * nch, nch)], dstv)
        z16 = jnp.zeros((16,), jnp.int32)
        for j in range(K // 16):
            srcv[nch, pl.ds(j * 16, 16)] = z16
        pltpu.sync_copy(zeros_hbm, acc_sh.at[pl.ds(s * rpw, rpw)])
        plsc.subcore_barrier()

        def chunk(i, carry):
            pltpu.async_copy(hs_hbm.at[srcv.at[i]], buf0, sem0).wait()
            pltpu.sync_copy(buf0, acc_sh.at[dstv.at[i]], add=True)
            return carry

        lax.fori_loop(0, nch, chunk, 0)
        plsc.subcore_barrier()
        pltpu.sync_copy(acc_sh.at[pl.ds(s * rpw, rpw)],
                        out_hbm.at[c, pl.ds(s * rpw, rpw)])

    return agg_kernel


def _row_spec(R, D):
    return pl.BlockSpec((R, D), lambda i: (i, 0))


def _full_spec(shape):
    nd = len(shape)
    return pl.BlockSpec(shape, lambda i, _nd=nd: (0,) * _nd)


def _scale_from_deg(degp_ref):
    deg = degp_ref[0, :, 0:1] + degp_ref[1, :, 0:1]
    return lax.rsqrt(deg + 1.0)


def _enc_call(features, degp, W_enc, b_enc, R):
    N, D_in = features.shape
    D = W_enc.shape[1]

    def body(x_ref, degp_ref, w_ref, b_ref, h_ref, hs_ref):
        h = jnp.dot(x_ref[...], w_ref[...],
                    preferred_element_type=jnp.float32) + b_ref[...]
        s = _scale_from_deg(degp_ref)
        h_ref[...] = h
        hs_ref[...] = h * s

    return pl.pallas_call(
        body,
        grid=(N // R,),
        in_specs=[
            _row_spec(R, D_in),
            pl.BlockSpec((NC, R, DEGW), lambda i: (0, i, 0)),
            _full_spec((D_in, D)),
            _full_spec((1, D)),
        ],
        out_specs=[_row_spec(R, D), _row_spec(R, D)],
        out_shape=[jax.ShapeDtypeStruct((N, D), jnp.float32)] * 2,
    )(features, degp, W_enc, b_enc[None, :])


def _comb_call(h, aggp, degp, W_top, W_bot, b, R):
    N, D = h.shape

    def body(h_ref, aggp_ref, degp_ref, wt_ref, wb_ref, b_ref,
             h2_ref, h2s_ref):
        s = _scale_from_deg(degp_ref)
        agg = (aggp_ref[0] + aggp_ref[1]) * s
        z = (jnp.dot(h_ref[...], wt_ref[...], preferred_element_type=jnp.float32)
             + jnp.dot(agg, wb_ref[...], preferred_element_type=jnp.float32)
             + b_ref[...])
        h2 = jnp.maximum(z, 0.0)
        h2_ref[...] = h2
        h2s_ref[...] = h2 * s

    return pl.pallas_call(
        body,
        grid=(N // R,),
        in_specs=[
            _row_spec(R, D),
            pl.BlockSpec((NC, R, D), lambda i: (0, i, 0)),
            pl.BlockSpec((NC, R, DEGW), lambda i: (0, i, 0)),
            _full_spec((D, D)),
            _full_spec((D, D)),
            _full_spec((1, D)),
        ],
        out_specs=[_row_spec(R, D), _row_spec(R, D)],
        out_shape=[jax.ShapeDtypeStruct((N, D), jnp.float32)] * 2,
    )(h, aggp, degp, W_top, W_bot, b[None, :])


def _comb_head_call(h, aggp, degp, W_top, W_bot, b, W_out, b_out, R):
    N, D = h.shape
    D_out = W_out.shape[1]

    def body(h_ref, aggp_ref, degp_ref, wt_ref, wb_ref, b_ref,
             wo_ref, bo_ref, out_ref):
        s = _scale_from_deg(degp_ref)
        agg = (aggp_ref[0] + aggp_ref[1]) * s
        z = (jnp.dot(h_ref[...], wt_ref[...], preferred_element_type=jnp.float32)
             + jnp.dot(agg, wb_ref[...], preferred_element_type=jnp.float32)
             + b_ref[...])
        h2 = jnp.maximum(z, 0.0)
        out_ref[...] = jnp.dot(h2, wo_ref[...],
                               preferred_element_type=jnp.float32) + bo_ref[...]

    return pl.pallas_call(
        body,
        grid=(N // R,),
        in_specs=[
            _row_spec(R, D),
            pl.BlockSpec((NC, R, D), lambda i: (0, i, 0)),
            pl.BlockSpec((NC, R, DEGW), lambda i: (0, i, 0)),
            _full_spec((D, D)),
            _full_spec((D, D)),
            _full_spec((1, D)),
            _full_spec((D, D_out)),
            _full_spec((1, D_out)),
        ],
        out_specs=pl.BlockSpec((R, D_out), lambda i: (i, 0)),
        out_shape=jax.ShapeDtypeStruct((N, D_out), jnp.float32),
    )(h, aggp, degp, W_top, W_bot, b[None, :], W_out, b_out[None, :])


def kernel(features, edge_list, W_enc, b_enc, W_comb, b_comb, W_out, b_out):
    N, D_in = features.shape
    E = edge_list.shape[0]
    D = W_enc.shape[1]
    layers = W_comb.shape[0]

    R = 1000
    N_pad = ((N + 16 * NS - 1) // (16 * NS)) * (16 * NS)
    E_pad = ((E + NW * K - 1) // (NW * K)) * (NW * K)

    src = edge_list[:, 0]
    dst = edge_list[:, 1]
    pad = E_pad - E
    src_p = jnp.concatenate([src, jnp.zeros((pad,), jnp.int32)])
    trash = N + jnp.arange(pad, dtype=jnp.int32) % (N_pad - N)
    dst_p = jnp.concatenate([dst, trash])
    src_p = src_p.reshape(E_pad // K, K)
    dst_p = dst_p.reshape(E_pad // K, K)

    rpw = N_pad // NS
    zeros_deg = jnp.zeros((rpw, DEGW), jnp.float32)
    ones_deg = jnp.ones((K, DEGW), jnp.float32)
    zeros_agg = jnp.zeros((rpw, D), jnp.float32)

    degp = _make_deg(N_pad, E_pad)(dst_p, ones_deg, zeros_deg)
    h, hs = _enc_call(features, degp, W_enc, b_enc, R)
    agg_fn = _make_agg(N_pad, E_pad, D)
    for l in range(layers - 1):
        aggp = agg_fn(hs, src_p, dst_p, zeros_agg)
        h, hs = _comb_call(h, aggp, degp,
                           W_comb[l, :D], W_comb[l, D:], b_comb[l], R)
    aggp = agg_fn(hs, src_p, dst_p, zeros_agg)
    return _comb_head_call(h, aggp, degp,
                           W_comb[layers - 1, :D], W_comb[layers - 1, D:],
                           b_comb[layers - 1], W_out, b_out, R)


# speedup vs baseline: 1.5493x; 1.1181x over previous
import functools

import jax
import jax.numpy as jnp
from jax import lax
from jax.experimental import pallas as pl
from jax.experimental.pallas import tpu as pltpu
from jax.experimental.pallas import tpu_sc as plsc

NC = 2
NS = 16
NW = NC * NS
K = 128
DEGW = 128


def _sc_mesh():
    return plsc.VectorSubcoreMesh(
        core_axis_name="c", subcore_axis_name="s",
        num_cores=NC, num_subcores=NS)


def _make_deg(N_pad, E_pad):
    epw = E_pad // NW
    nch = epw // K
    rpw = N_pad // NS

    @functools.partial(
        pl.kernel,
        out_type=jax.ShapeDtypeStruct((NC, N_pad, DEGW), jnp.float32),
        mesh=_sc_mesh(),
        scratch_types=[
            pltpu.VMEM((nch, K), jnp.int32),
            pltpu.VMEM((K, DEGW), jnp.float32),
            pltpu.VMEM_SHARED((N_pad, DEGW), jnp.float32),
        ],
    )
    def deg_kernel(dst_hbm, ones_hbm, zeros_hbm, out_hbm, dstv, ones_v, acc_sh):
        c = lax.axis_index("c")
        s = lax.axis_index("s")
        wid = c * NS + s
        pltpu.sync_copy(ones_hbm, ones_v)
        pltpu.sync_copy(dst_hbm.at[pl.ds(wid * nch, nch)], dstv)
        pltpu.sync_copy(zeros_hbm, acc_sh.at[pl.ds(s * rpw, rpw)])
        plsc.subcore_barrier()

        def chunk(i, carry):
            pltpu.sync_copy(ones_v, acc_sh.at[dstv.at[i]], add=True)
            return carry

        lax.fori_loop(0, nch, chunk, 0)
        plsc.subcore_barrier()
        pltpu.sync_copy(acc_sh.at[pl.ds(s * rpw, rpw)],
                        out_hbm.at[c, pl.ds(s * rpw, rpw)])

    return deg_kernel


def _make_agg(N_pad, D, n0, n1):
    rpw = N_pad // NS

    @functools.partial(
        pl.kernel,
        out_type=jax.ShapeDtypeStruct((NC, N_pad, D), jnp.float32),
        mesh=_sc_mesh(),
        scratch_types=[
            pltpu.VMEM((n0, K), jnp.int32),
            pltpu.VMEM((n0, K), jnp.int32),
            pltpu.VMEM((K, D), jnp.float32),
            pltpu.VMEM_SHARED((N_pad, D), jnp.float32),
            pltpu.SemaphoreType.DMA,
        ],
    )
    def agg_kernel(hs_hbm, src_hbm, dst_hbm, zeros_hbm, out_hbm,
                   srcv, dstv, buf0, acc_sh, sem0):
        c = lax.axis_index("c")
        s = lax.axis_index("s")
        base_row = jnp.where(c == 0, s * n0, NS * n0 + s * n1)
        nch_c = jnp.where(c == 0, n0, n1)
        pltpu.sync_copy(src_hbm.at[pl.ds(base_row, n0)], srcv)
        pltpu.sync_copy(dst_hbm.at[pl.ds(base_row, n0)], dstv)
        pltpu.sync_copy(zeros_hbm, acc_sh.at[pl.ds(s * rpw, rpw)])
        plsc.subcore_barrier()

        def chunk(i, carry):
            pltpu.async_copy(hs_hbm.at[srcv.at[i]], buf0, sem0).wait()
            pltpu.sync_copy(buf0, acc_sh.at[dstv.at[i]], add=True)
            return carry

        lax.fori_loop(0, nch_c, chunk, 0)
        plsc.subcore_barrier()
        pltpu.sync_copy(acc_sh.at[pl.ds(s * rpw, rpw)],
                        out_hbm.at[c, pl.ds(s * rpw, rpw)])

    return agg_kernel


def _row_spec(R, D):
    return pl.BlockSpec((R, D), lambda i: (i, 0))


def _full_spec(shape):
    nd = len(shape)
    return pl.BlockSpec(shape, lambda i, _nd=nd: (0,) * _nd)


def _scale_from_deg(degp_ref):
    deg = degp_ref[0, :, 0:1] + degp_ref[1, :, 0:1]
    return lax.rsqrt(deg + 1.0)


def _enc_call(features, degp, W_enc, b_enc, R):
    N, D_in = features.shape
    D = W_enc.shape[1]

    def body(x_ref, degp_ref, w_ref, b_ref, h_ref, hs_ref):
        h = jnp.dot(x_ref[...], w_ref[...],
                    preferred_element_type=jnp.float32) + b_ref[...]
        s = _scale_from_deg(degp_ref)
        h_ref[...] = h
        hs_ref[...] = h * s

    return pl.pallas_call(
        body,
        grid=(N // R,),
        in_specs=[
            _row_spec(R, D_in),
            pl.BlockSpec((NC, R, DEGW), lambda i: (0, i, 0)),
            _full_spec((D_in, D)),
            _full_spec((1, D)),
        ],
        out_specs=[_row_spec(R, D), _row_spec(R, D)],
        out_shape=[jax.ShapeDtypeStruct((N, D), jnp.float32)] * 2,
    )(features, degp, W_enc, b_enc[None, :])


def _comb_call(h, aggp, degp, W_top, W_bot, b, R):
    N, D = h.shape

    def body(h_ref, aggp_ref, degp_ref, wt_ref, wb_ref, b_ref,
             h2_ref, h2s_ref):
        s = _scale_from_deg(degp_ref)
        agg = (aggp_ref[0] + aggp_ref[1]) * s
        z = (jnp.dot(h_ref[...], wt_ref[...], preferred_element_type=jnp.float32)
             + jnp.dot(agg, wb_ref[...], preferred_element_type=jnp.float32)
             + b_ref[...])
        h2 = jnp.maximum(z, 0.0)
        h2_ref[...] = h2
        h2s_ref[...] = h2 * s

    return pl.pallas_call(
        body,
        grid=(N // R,),
        in_specs=[
            _row_spec(R, D),
            pl.BlockSpec((NC, R, D), lambda i: (0, i, 0)),
            pl.BlockSpec((NC, R, DEGW), lambda i: (0, i, 0)),
            _full_spec((D, D)),
            _full_spec((D, D)),
            _full_spec((1, D)),
        ],
        out_specs=[_row_spec(R, D), _row_spec(R, D)],
        out_shape=[jax.ShapeDtypeStruct((N, D), jnp.float32)] * 2,
    )(h, aggp, degp, W_top, W_bot, b[None, :])


def _comb_head_call(h, aggp, degp, W_top, W_bot, b, W_out, b_out, R):
    N, D = h.shape
    D_out = W_out.shape[1]

    def body(h_ref, aggp_ref, degp_ref, wt_ref, wb_ref, b_ref,
             wo_ref, bo_ref, out_ref):
        s = _scale_from_deg(degp_ref)
        agg = (aggp_ref[0] + aggp_ref[1]) * s
        z = (jnp.dot(h_ref[...], wt_ref[...], preferred_element_type=jnp.float32)
             + jnp.dot(agg, wb_ref[...], preferred_element_type=jnp.float32)
             + b_ref[...])
        h2 = jnp.maximum(z, 0.0)
        out_ref[...] = jnp.dot(h2, wo_ref[...],
                               preferred_element_type=jnp.float32) + bo_ref[...]

    return pl.pallas_call(
        body,
        grid=(N // R,),
        in_specs=[
            _row_spec(R, D),
            pl.BlockSpec((NC, R, D), lambda i: (0, i, 0)),
            pl.BlockSpec((NC, R, DEGW), lambda i: (0, i, 0)),
            _full_spec((D, D)),
            _full_spec((D, D)),
            _full_spec((1, D)),
            _full_spec((D, D_out)),
            _full_spec((1, D_out)),
        ],
        out_specs=pl.BlockSpec((R, D_out), lambda i: (i, 0)),
        out_shape=jax.ShapeDtypeStruct((N, D_out), jnp.float32),
    )(h, aggp, degp, W_top, W_bot, b[None, :], W_out, b_out[None, :])


def kernel(features, edge_list, W_enc, b_enc, W_comb, b_comb, W_out, b_out):
    N, D_in = features.shape
    E = edge_list.shape[0]
    D = W_enc.shape[1]
    layers = W_comb.shape[0]

    R = 1000
    N_pad = ((N + 16 * NS - 1) // (16 * NS)) * (16 * NS)
    E_pad = ((E + NW * K - 1) // (NW * K)) * (NW * K)

    n0 = 56
    n1 = E_pad // K // NS - n0
    alloc_rows = E_pad // K + (n0 - n1)

    src = edge_list[:, 0]
    dst = edge_list[:, 1]
    pad = alloc_rows * K - E
    src_p = jnp.concatenate([src, jnp.zeros((pad,), jnp.int32)])
    trash = N + jnp.arange(pad, dtype=jnp.int32) % (N_pad - N)
    dst_p = jnp.concatenate([dst, trash])
    src_p = src_p.reshape(alloc_rows, K)
    dst_p = dst_p.reshape(alloc_rows, K)

    rpw = N_pad // NS
    zeros_deg = jnp.zeros((rpw, DEGW), jnp.float32)
    ones_deg = jnp.ones((K, DEGW), jnp.float32)
    zeros_agg = jnp.zeros((rpw, D), jnp.float32)

    degp = _make_deg(N_pad, E_pad)(dst_p, ones_deg, zeros_deg)
    h, hs = _enc_call(features, degp, W_enc, b_enc, R)
    agg_fn = _make_agg(N_pad, D, n0, n1)
    for l in range(layers - 1):
        aggp = agg_fn(hs, src_p, dst_p, zeros_agg)
        h, hs = _comb_call(h, aggp, degp,
                           W_comb[l, :D], W_comb[l, D:], b_comb[l], R)
    aggp = agg_fn(hs, src_p, dst_p, zeros_agg)
    return _comb_head_call(h, aggp, degp,
                           W_comb[layers - 1, :D], W_comb[layers - 1, D:],
                           b_comb[layers - 1], W_out, b_out, R)


# speedup vs baseline: 1.7414x; 1.1240x over previous
import functools

import jax
import jax.numpy as jnp
from jax import lax
from jax.experimental import pallas as pl
from jax.experimental.pallas import tpu as pltpu
from jax.experimental.pallas import tpu_sc as plsc

NC = 2
NS = 16
NW = NC * NS
K = 128
DEGW = 128


def _sc_mesh():
    return plsc.VectorSubcoreMesh(
        core_axis_name="c", subcore_axis_name="s",
        num_cores=NC, num_subcores=NS)


def _make_deg(N_pad, E_pad):
    epw = E_pad // NW
    nch = epw // K
    rpw = N_pad // NS

    @functools.partial(
        pl.kernel,
        out_type=jax.ShapeDtypeStruct((NC, N_pad, DEGW), jnp.float32),
        mesh=_sc_mesh(),
        scratch_types=[
            pltpu.VMEM((nch, K), jnp.int32),
            pltpu.VMEM((K, DEGW), jnp.float32),
            pltpu.VMEM_SHARED((N_pad, DEGW), jnp.float32),
        ],
    )
    def deg_kernel(dst_hbm, ones_hbm, zeros_hbm, out_hbm, dstv, ones_v, acc_sh):
        c = lax.axis_index("c")
        s = lax.axis_index("s")
        wid = c * NS + s
        pltpu.sync_copy(ones_hbm, ones_v)
        pltpu.sync_copy(dst_hbm.at[pl.ds(wid * nch, nch)], dstv)
        pltpu.sync_copy(zeros_hbm, acc_sh.at[pl.ds(s * rpw, rpw)])
        plsc.subcore_barrier()

        def chunk(i, carry):
            pltpu.sync_copy(ones_v, acc_sh.at[dstv.at[i]], add=True)
            return carry

        lax.fori_loop(0, nch, chunk, 0)
        plsc.subcore_barrier()
        pltpu.sync_copy(acc_sh.at[pl.ds(s * rpw, rpw)],
                        out_hbm.at[c, pl.ds(s * rpw, rpw)])

    return deg_kernel


def _make_agg(N_pad, D, n0, n1):
    rpw = N_pad // NS

    @functools.partial(
        pl.kernel,
        out_type=jax.ShapeDtypeStruct((NC, N_pad, D), jnp.float32),
        mesh=_sc_mesh(),
        scratch_types=[
            pltpu.VMEM((n0, K), jnp.int32),
            pltpu.VMEM((n0, K), jnp.int32),
            pltpu.VMEM((K, D), jnp.float32),
            pltpu.VMEM_SHARED((N_pad, D), jnp.float32),
            pltpu.SemaphoreType.DMA,
        ],
    )
    def agg_kernel(hs_hbm, src_hbm, dst_hbm, zeros_hbm, out_hbm,
                   srcv, dstv, buf0, acc_sh, sem0):
        c = lax.axis_index("c")
        s = lax.axis_index("s")
        base_row = jnp.where(c == 0, s * n0, NS * n0 + s * n1)
        nch_c = jnp.where(c == 0, n0, n1)
        pltpu.sync_copy(src_hbm.at[pl.ds(base_row, n0)], srcv)
        pltpu.sync_copy(dst_hbm.at[pl.ds(base_row, n0)], dstv)
        pltpu.sync_copy(zeros_hbm, acc_sh.at[pl.ds(s * rpw, rpw)])
        plsc.subcore_barrier()

        def chunk(i, carry):
            pltpu.async_copy(hs_hbm.at[srcv.at[i]], buf0, sem0).wait()
            pltpu.sync_copy(buf0, acc_sh.at[dstv.at[i]], add=True)
            return carry

        lax.fori_loop(0, nch_c, chunk, 0)
        plsc.subcore_barrier()
        pltpu.sync_copy(acc_sh.at[pl.ds(s * rpw, rpw)],
                        out_hbm.at[c, pl.ds(s * rpw, rpw)])

    return agg_kernel


def _row_spec(R, D):
    return pl.BlockSpec((R, D), lambda i: (i, 0))


def _full_spec(shape):
    nd = len(shape)
    return pl.BlockSpec(shape, lambda i, _nd=nd: (0,) * _nd)


def _scale_from_deg(degp_ref):
    deg = degp_ref[0, :, 0:1] + degp_ref[1, :, 0:1]
    return lax.rsqrt(deg + 1.0)


def _enc_call(features, degp, W_enc, b_enc, R):
    N, D_in = features.shape
    D = W_enc.shape[1]

    def body(x_ref, degp_ref, w_ref, b_ref, h_ref, hs_ref):
        h = jnp.dot(x_ref[...], w_ref[...],
                    preferred_element_type=jnp.float32) + b_ref[...]
        s = _scale_from_deg(degp_ref)
        h_ref[...] = h
        hs_ref[...] = h * s

    return pl.pallas_call(
        body,
        grid=(N // R,),
        in_specs=[
            _row_spec(R, D_in),
            pl.BlockSpec((NC, R, DEGW), lambda i: (0, i, 0)),
            _full_spec((D_in, D)),
            _full_spec((1, D)),
        ],
        out_specs=[_row_spec(R, D), _row_spec(R, D)],
        out_shape=[jax.ShapeDtypeStruct((N, D), jnp.float32)] * 2,
    )(features, degp, W_enc, b_enc[None, :])


def _comb_call(h, aggp, degp, W_top, W_bot, b, R):
    N, D = h.shape

    def body(h_ref, aggp_ref, degp_ref, wt_ref, wb_ref, b_ref,
             h2_ref, h2s_ref):
        s = _scale_from_deg(degp_ref)
        agg = (aggp_ref[0] + aggp_ref[1]) * s
        z = (jnp.dot(h_ref[...], wt_ref[...], preferred_element_type=jnp.float32)
             + jnp.dot(agg, wb_ref[...], preferred_element_type=jnp.float32)
             + b_ref[...])
        h2 = jnp.maximum(z, 0.0)
        h2_ref[...] = h2
        h2s_ref[...] = h2 * s

    return pl.pallas_call(
        body,
        grid=(N // R,),
        in_specs=[
            _row_spec(R, D),
            pl.BlockSpec((NC, R, D), lambda i: (0, i, 0)),
            pl.BlockSpec((NC, R, DEGW), lambda i: (0, i, 0)),
            _full_spec((D, D)),
            _full_spec((D, D)),
            _full_spec((1, D)),
        ],
        out_specs=[_row_spec(R, D), _row_spec(R, D)],
        out_shape=[jax.ShapeDtypeStruct((N, D), jnp.float32)] * 2,
    )(h, aggp, degp, W_top, W_bot, b[None, :])


def _comb_head_call(h, aggp, degp, W_top, W_bot, b, W_out, b_out, R):
    N, D = h.shape
    D_out = W_out.shape[1]

    def body(h_ref, aggp_ref, degp_ref, wt_ref, wb_ref, b_ref,
             wo_ref, bo_ref, out_ref):
        s = _scale_from_deg(degp_ref)
        agg = (aggp_ref[0] + aggp_ref[1]) * s
        z = (jnp.dot(h_ref[...], wt_ref[...], preferred_element_type=jnp.float32)
             + jnp.dot(agg, wb_ref[...], preferred_element_type=jnp.float32)
             + b_ref[...])
        h2 = jnp.maximum(z, 0.0)
        out_ref[...] = jnp.dot(h2, wo_ref[...],
                               preferred_element_type=jnp.float32) + bo_ref[...]

    return pl.pallas_call(
        body,
        grid=(N // R,),
        in_specs=[
            _row_spec(R, D),
            pl.BlockSpec((NC, R, D), lambda i: (0, i, 0)),
            pl.BlockSpec((NC, R, DEGW), lambda i: (0, i, 0)),
            _full_spec((D, D)),
            _full_spec((D, D)),
            _full_spec((1, D)),
            _full_spec((D, D_out)),
            _full_spec((1, D_out)),
        ],
        out_specs=pl.BlockSpec((R, D_out), lambda i: (i, 0)),
        out_shape=jax.ShapeDtypeStruct((N, D_out), jnp.float32),
    )(h, aggp, degp, W_top, W_bot, b[None, :], W_out, b_out[None, :])


def kernel(features, edge_list, W_enc, b_enc, W_comb, b_comb, W_out, b_out):
    N, D_in = features.shape
    E = edge_list.shape[0]
    D = W_enc.shape[1]
    layers = W_comb.shape[0]

    R = 1000
    N_pad = ((N + 16 * NS - 1) // (16 * NS)) * (16 * NS)
    E_pad = ((E + NW * K - 1) // (NW * K)) * (NW * K)

    n0 = 64
    n1 = E_pad // K // NS - n0
    alloc_rows = E_pad // K + (n0 - n1)

    src = edge_list[:, 0]
    dst = edge_list[:, 1]
    pad = alloc_rows * K - E
    src_p = jnp.concatenate([src, jnp.zeros((pad,), jnp.int32)])
    trash = N + jnp.arange(pad, dtype=jnp.int32) % (N_pad - N)
    dst_p = jnp.concatenate([dst, trash])
    src_p = src_p.reshape(alloc_rows, K)
    dst_p = dst_p.reshape(alloc_rows, K)

    rpw = N_pad // NS
    zeros_deg = jnp.zeros((rpw, DEGW), jnp.float32)
    ones_deg = jnp.ones((K, DEGW), jnp.float32)
    zeros_agg = jnp.zeros((rpw, D), jnp.float32)

    degp = _make_deg(N_pad, E_pad)(dst_p, ones_deg, zeros_deg)
    h, hs = _enc_call(features, degp, W_enc, b_enc, R)
    agg_fn = _make_agg(N_pad, D, n0, n1)
    for l in range(layers - 1):
        aggp = agg_fn(hs, src_p, dst_p, zeros_agg)
        h, hs = _comb_call(h, aggp, degp,
                           W_comb[l, :D], W_comb[l, D:], b_comb[l], R)
    aggp = agg_fn(hs, src_p, dst_p, zeros_agg)
    return _comb_head_call(h, aggp, degp,
                           W_comb[layers - 1, :D], W_comb[layers - 1, D:],
                           b_comb[layers - 1], W_out, b_out, R)


# speedup vs baseline: 1.7497x; 1.0047x over previous
import functools

import jax
import jax.numpy as jnp
from jax import lax
from jax.experimental import pallas as pl
from jax.experimental.pallas import tpu as pltpu
from jax.experimental.pallas import tpu_sc as plsc

NC = 2
NS = 16
NW = NC * NS
K = 128
DEGW = 128


def _sc_mesh():
    return plsc.VectorSubcoreMesh(
        core_axis_name="c", subcore_axis_name="s",
        num_cores=NC, num_subcores=NS)


def _make_deg(N_pad, E_pad):
    epw = E_pad // NW
    nch = epw // K
    rpw = N_pad // NS

    @functools.partial(
        pl.kernel,
        out_type=jax.ShapeDtypeStruct((NC, N_pad, DEGW), jnp.float32),
        mesh=_sc_mesh(),
        scratch_types=[
            pltpu.VMEM((nch, K), jnp.int32),
            pltpu.VMEM((K, DEGW), jnp.float32),
            pltpu.VMEM_SHARED((N_pad, DEGW), jnp.float32),
        ],
    )
    def deg_kernel(dst_hbm, ones_hbm, zeros_hbm, out_hbm, dstv, ones_v, acc_sh):
        c = lax.axis_index("c")
        s = lax.axis_index("s")
        wid = c * NS + s
        pltpu.sync_copy(ones_hbm, ones_v)
        pltpu.sync_copy(dst_hbm.at[pl.ds(wid * nch, nch)], dstv)
        pltpu.sync_copy(zeros_hbm, acc_sh.at[pl.ds(s * rpw, rpw)])
        plsc.subcore_barrier()

        def chunk(i, carry):
            pltpu.sync_copy(ones_v, acc_sh.at[dstv.at[i]], add=True)
            return carry

        lax.fori_loop(0, nch, chunk, 0)
        plsc.subcore_barrier()
        pltpu.sync_copy(acc_sh.at[pl.ds(s * rpw, rpw)],
                        out_hbm.at[c, pl.ds(s * rpw, rpw)])

    return deg_kernel


def _make_agg(N_pad, D, n0, n1):
    rpw = N_pad // NS

    @functools.partial(
        pl.kernel,
        out_type=jax.ShapeDtypeStruct((NC, N_pad, D), jnp.float32),
        mesh=_sc_mesh(),
        scratch_types=[
            pltpu.VMEM((n0, K), jnp.int32),
            pltpu.VMEM((n0, K), jnp.int32),
            pltpu.VMEM((K, D), jnp.float32),
            pltpu.VMEM_SHARED((N_pad, D), jnp.float32),
            pltpu.SemaphoreType.DMA,
        ],
    )
    def agg_kernel(hs_hbm, src_hbm, dst_hbm, zeros_hbm, out_hbm,
                   srcv, dstv, buf0, acc_sh, sem0):
        c = lax.axis_index("c")
        s = lax.axis_index("s")
        base_row = jnp.where(c == 0, s * n0, NS * n0 + s * n1)
        nch_c = jnp.where(c == 0, n0, n1)
        pltpu.sync_copy(src_hbm.at[pl.ds(base_row, n0)], srcv)
        pltpu.sync_copy(dst_hbm.at[pl.ds(base_row, n0)], dstv)
        pltpu.sync_copy(zeros_hbm, acc_sh.at[pl.ds(s * rpw, rpw)])
        plsc.subcore_barrier()

        def chunk(i, carry):
            pltpu.async_copy(hs_hbm.at[srcv.at[i]], buf0, sem0).wait()
            pltpu.sync_copy(buf0, acc_sh.at[dstv.at[i]], add=True)
            return carry

        lax.fori_loop(0, nch_c, chunk, 0)
        plsc.subcore_barrier()
        pltpu.sync_copy(acc_sh.at[pl.ds(s * rpw, rpw)],
                        out_hbm.at[c, pl.ds(s * rpw, rpw)])

    return agg_kernel


def _row_spec(R, D):
    return pl.BlockSpec((R, D), lambda i: (i, 0))


def _full_spec(shape):
    nd = len(shape)
    return pl.BlockSpec(shape, lambda i, _nd=nd: (0,) * _nd)


def _scale_from_deg(degp_ref):
    deg = degp_ref[0, :, 0:1] + degp_ref[1, :, 0:1]
    return lax.rsqrt(deg + 1.0)


def _enc_call(features, degp, W_enc, b_enc, R):
    N, D_in = features.shape
    D = W_enc.shape[1]

    def body(x_ref, degp_ref, w_ref, b_ref, h_ref, hs_ref):
        h = jnp.dot(x_ref[...], w_ref[...],
                    preferred_element_type=jnp.float32) + b_ref[...]
        s = _scale_from_deg(degp_ref)
        h_ref[...] = h
        hs_ref[...] = h * s

    return pl.pallas_call(
        body,
        grid=(N // R,),
        in_specs=[
            _row_spec(R, D_in),
            pl.BlockSpec((NC, R, DEGW), lambda i: (0, i, 0)),
            _full_spec((D_in, D)),
            _full_spec((1, D)),
        ],
        out_specs=[_row_spec(R, D), _row_spec(R, D)],
        out_shape=[jax.ShapeDtypeStruct((N, D), jnp.float32)] * 2,
    )(features, degp, W_enc, b_enc[None, :])


def _comb_call(h, aggp, degp, W_top, W_bot, b, R):
    N, D = h.shape

    def body(h_ref, aggp_ref, degp_ref, wt_ref, wb_ref, b_ref,
             h2_ref, h2s_ref):
        s = _scale_from_deg(degp_ref)
        agg = (aggp_ref[0] + aggp_ref[1]) * s
        z = (jnp.dot(h_ref[...], wt_ref[...], preferred_element_type=jnp.float32)
             + jnp.dot(agg, wb_ref[...], preferred_element_type=jnp.float32)
             + b_ref[...])
        h2 = jnp.maximum(z, 0.0)
        h2_ref[...] = h2
        h2s_ref[...] = h2 * s

    return pl.pallas_call(
        body,
        grid=(N // R,),
        in_specs=[
            _row_spec(R, D),
            pl.BlockSpec((NC, R, D), lambda i: (0, i, 0)),
            pl.BlockSpec((NC, R, DEGW), lambda i: (0, i, 0)),
            _full_spec((D, D)),
            _full_spec((D, D)),
            _full_spec((1, D)),
        ],
        out_specs=[_row_spec(R, D), _row_spec(R, D)],
        out_shape=[jax.ShapeDtypeStruct((N, D), jnp.float32)] * 2,
    )(h, aggp, degp, W_top, W_bot, b[None, :])


def _comb_head_call(h, aggp, degp, W_top, W_bot, b, W_out, b_out, R):
    N, D = h.shape
    D_out = W_out.shape[1]

    def body(h_ref, aggp_ref, degp_ref, wt_ref, wb_ref, b_ref,
             wo_ref, bo_ref, out_ref):
        s = _scale_from_deg(degp_ref)
        agg = (aggp_ref[0] + aggp_ref[1]) * s
        z = (jnp.dot(h_ref[...], wt_ref[...], preferred_element_type=jnp.float32)
             + jnp.dot(agg, wb_ref[...], preferred_element_type=jnp.float32)
             + b_ref[...])
        h2 = jnp.maximum(z, 0.0)
        out_ref[...] = jnp.dot(h2, wo_ref[...],
                               preferred_element_type=jnp.float32) + bo_ref[...]

    return pl.pallas_call(
        body,
        grid=(N // R,),
        in_specs=[
            _row_spec(R, D),
            pl.BlockSpec((NC, R, D), lambda i: (0, i, 0)),
            pl.BlockSpec((NC, R, DEGW), lambda i: (0, i, 0)),
            _full_spec((D, D)),
            _full_spec((D, D)),
            _full_spec((1, D)),
            _full_spec((D, D_out)),
            _full_spec((1, D_out)),
        ],
        out_specs=pl.BlockSpec((R, D_out), lambda i: (i, 0)),
        out_shape=jax.ShapeDtypeStruct((N, D_out), jnp.float32),
    )(h, aggp, degp, W_top, W_bot, b[None, :], W_out, b_out[None, :])


def kernel(features, edge_list, W_enc, b_enc, W_comb, b_comb, W_out, b_out):
    N, D_in = features.shape
    E = edge_list.shape[0]
    D = W_enc.shape[1]
    layers = W_comb.shape[0]

    R = 1000
    N_pad = ((N + 16 * NS - 1) // (16 * NS)) * (16 * NS)
    E_pad = ((E + NW * K - 1) // (NW * K)) * (NW * K)

    n0 = 72
    n1 = E_pad // K // NS - n0
    alloc_rows = E_pad // K + (n0 - n1)

    src = edge_list[:, 0]
    dst = edge_list[:, 1]
    pad = alloc_rows * K - E
    src_p = jnp.concatenate([src, jnp.zeros((pad,), jnp.int32)])
    trash = N + jnp.arange(pad, dtype=jnp.int32) % (N_pad - N)
    dst_p = jnp.concatenate([dst, trash])
    src_p = src_p.reshape(alloc_rows, K)
    dst_p = dst_p.reshape(alloc_rows, K)

    rpw = N_pad // NS
    zeros_deg = jnp.zeros((rpw, DEGW), jnp.float32)
    ones_deg = jnp.ones((K, DEGW), jnp.float32)
    zeros_agg = jnp.zeros((rpw, D), jnp.float32)

    degp = _make_deg(N_pad, E_pad)(dst_p, ones_deg, zeros_deg)
    h, hs = _enc_call(features, degp, W_enc, b_enc, R)
    agg_fn = _make_agg(N_pad, D, n0, n1)
    for l in range(layers - 1):
        aggp = agg_fn(hs, src_p, dst_p, zeros_agg)
        h, hs = _comb_call(h, aggp, degp,
                           W_comb[l, :D], W_comb[l, D:], b_comb[l], R)
    aggp = agg_fn(hs, src_p, dst_p, zeros_agg)
    return _comb_head_call(h, aggp, degp,
                           W_comb[layers - 1, :D], W_comb[layers - 1, D:],
                           b_comb[layers - 1], W_out, b_out, R)
